# baseline (speedup 1.0000x reference)
import functools

import jax
import jax.numpy as jnp
import numpy as np
from jax.experimental import pallas as pl

PARTITIONS = 8
D_H = 128
D = 128
OBS = 8
B = 1024
NEI = 64

BB = 8


def _fused_kernel(x_ego_ref, x_nei_ref, W_te_ref, b_te_ref, W1_ref, b1_ref,
                  W2_ref, b2_ref, W3_ref, b3_ref, W_ce_ref, b_ce_ref,
                  f_re_ref, re_ref):
    bb = x_ego_ref.shape[0]
    x_ego = x_ego_ref[...]
    x_nei = x_nei_ref[...]
    ego_last = x_ego[:, OBS - 1:OBS, :]

    w0 = W_te_ref[0:1, :].reshape(1, 1, 1, D_H)
    w1 = W_te_ref[1:2, :].reshape(1, 1, 1, D_H)
    bte = b_te_ref[...].reshape(1, 1, 1, D_H)

    x_ego_pure = x_ego - ego_last
    ex = x_ego_pure[:, None, :, 0:1]
    ey = x_ego_pure[:, None, :, 1:2]
    f_ego = jax.nn.relu(ex * w0 + ey * w1 + bte)

    x_nei_pure = x_nei - x_nei[:, :, OBS - 1:OBS, :]
    nx = x_nei_pure[..., 0:1]
    ny = x_nei_pure[..., 1:2]
    f_nei = jax.nn.relu(nx * w0 + ny * w1 + bte)

    f = (f_ego * f_nei).reshape(bb * NEI * OBS, D_H)

    h = jax.nn.relu(
        jnp.dot(f, W1_ref[...], preferred_element_type=jnp.float32)
        + b1_ref[...])
    h = jax.nn.relu(
        jnp.dot(h, W2_ref[...], preferred_element_type=jnp.float32)
        + b2_ref[...])
    f_re = jax.nn.relu(
        jnp.dot(h, W3_ref[...], preferred_element_type=jnp.float32)
        + b3_ref[...])
    f_re_ref[...] = f_re.reshape(bb, NEI, OBS, D // 2)

    p = x_nei + ego_last[:, None, :, :] - x_ego[:, None, :, :]
    px = p[..., 0]
    py = p[..., 1]
    dist = jnp.sqrt(px * px + py * py)
    ang = jnp.arctan2(px, py) % (2.0 * np.pi)
    idx = (ang / (2.0 * np.pi / PARTITIONS)).astype(jnp.int32)
    valid = (jnp.abs(px + py) > 1e-6) & (dist > 0.005)
    idx = jnp.where(valid, idx, -1)

    c = jax.lax.broadcasted_iota(jnp.int32, (1, 1, 1, OBS * PARTITIONS), 3)
    t = jax.lax.broadcasted_iota(jnp.int32, (1, 1, OBS, 1), 2)
    M = ((c // PARTITIONS == t) & (c % PARTITIONS == idx[..., None]))
    M = M.astype(jnp.float32)

    ones = jnp.ones_like(dist[..., None])
    zeros = jnp.zeros(dist.shape + (D - D // 2 - 3,), jnp.float32)
    F = jnp.concatenate(
        [f_re.reshape(bb, NEI, OBS, D // 2),
         dist[..., None], ang[..., None], ones, zeros], axis=-1)

    Mf = M.reshape(bb, NEI * OBS, OBS * PARTITIONS)
    Ff = F.reshape(bb, NEI * OBS, D)
    out = jax.lax.dot_general(
        Mf, Ff, (((1,), (1,)), ((0,), (0,))),
        preferred_element_type=jnp.float32)

    n = out[..., D // 2 + 2:D // 2 + 3] + 0.0001
    re_part = out[..., :D // 2] / n
    d_mean = out[..., D // 2:D // 2 + 1] / n
    a_mean = out[..., D // 2 + 1:D // 2 + 2] / n

    wc0 = W_ce_ref[0:1, :].reshape(1, 1, D // 2)
    wc1 = W_ce_ref[1:2, :].reshape(1, 1, D // 2)
    f_pos = jax.nn.relu(d_mean * wc0 + a_mean * wc1
                        + b_ce_ref[...].reshape(1, 1, D // 2))

    re_ref[...] = jnp.concatenate([re_part, f_pos], axis=-1)


@jax.jit
def kernel(x_ego_2d, x_nei_2d, W_te, b_te, W1, b1, W2, b2, W3, b3, W_ce, b_ce):
    grid = (B // BB,)
    b_te2 = b_te.reshape(1, D_H)
    b1_2 = b1.reshape(1, D_H)
    b2_2 = b2.reshape(1, D_H)
    b3_2 = b3.reshape(1, D // 2)
    b_ce2 = b_ce.reshape(1, D // 2)

    def rep(shape):
        return pl.BlockSpec(shape, lambda i: (0,) * len(shape))

    f_re, re_flat = pl.pallas_call(
        _fused_kernel,
        grid=grid,
        in_specs=[
            pl.BlockSpec((BB, OBS, 2), lambda i: (i, 0, 0)),
            pl.BlockSpec((BB, NEI, OBS, 2), lambda i: (i, 0, 0, 0)),
            rep((2, D_H)), rep((1, D_H)),
            rep((D_H, D_H)), rep((1, D_H)),
            rep((D_H, D_H)), rep((1, D_H)),
            rep((D_H, D // 2)), rep((1, D // 2)),
            rep((2, D // 2)), rep((1, D // 2)),
        ],
        out_specs=[
            pl.BlockSpec((BB, NEI, OBS, D // 2), lambda i: (i, 0, 0, 0)),
            pl.BlockSpec((BB, OBS * PARTITIONS, D), lambda i: (i, 0, 0)),
        ],
        out_shape=[
            jax.ShapeDtypeStruct((B, NEI, OBS, D // 2), jnp.float32),
            jax.ShapeDtypeStruct((B, OBS * PARTITIONS, D), jnp.float32),
        ],
    )(x_ego_2d, x_nei_2d, W_te, b_te2, W1, b1_2, W2, b2_2, W3, b3_2,
      W_ce, b_ce2)

    re_matrix = re_flat.reshape(B, OBS, PARTITIONS, D)
    return (re_matrix, f_re)


# speedup vs baseline: 1.9464x; 1.9464x over previous
import jax
import jax.numpy as jnp
import numpy as np
from jax.experimental import pallas as pl
from jax.experimental.pallas import tpu as pltpu

PARTITIONS = 8
D_H = 128
D = 128
OBS = 8
B = 1024
NEI = 64

BB = 8


def _fused_kernel(x_ego_ref, x_nei_ref, x_nei_T_ref, W_te_ref, b_te_ref,
                  W1_ref, b1_ref, W2_ref, b2_ref, W3_ref, b3_ref,
                  W_ce_ref, b_ce_ref, f_re_ref, re_ref):
    bb = x_ego_ref.shape[0]
    f32 = jnp.float32
    x_ego = x_ego_ref[...]
    ego_last = x_ego[:, OBS - 1:OBS, :]

    x_nei = x_nei_ref[...]
    x_nei_pure = x_nei - x_nei[:, :, OBS - 1:OBS, :]
    f_nei = jax.nn.relu(
        jnp.dot(x_nei_pure.reshape(bb * NEI * OBS, 2), W_te_ref[...],
                preferred_element_type=f32) + b_te_ref[...])
    f_nei = f_nei.reshape(bb, NEI, OBS, D_H)

    x_ego_pure = x_ego - ego_last
    f_ego = jax.nn.relu(
        jnp.dot(x_ego_pure.reshape(bb * OBS, 2), W_te_ref[...],
                preferred_element_type=f32) + b_te_ref[...])
    f_ego = f_ego.reshape(bb, 1, OBS, D_H)

    f = (f_ego * f_nei).reshape(bb * NEI * OBS, D_H)

    h = jax.nn.relu(
        jnp.dot(f, W1_ref[...], preferred_element_type=f32) + b1_ref[...])
    h = jax.nn.relu(
        jnp.dot(h, W2_ref[...], preferred_element_type=f32) + b2_ref[...])
    f_re = jax.nn.relu(
        jnp.dot(h, W3_ref[...], preferred_element_type=f32) + b3_ref[...])
    f_re_ref[...] = f_re.reshape(bb, NEI, OBS, D // 2)

    xT = x_nei_T_ref[...]
    egoT = x_ego
    ego_dx = (ego_last[:, :, 0] - egoT[:, :, 0])[..., None]
    ego_dy = (ego_last[:, :, 1] - egoT[:, :, 1])[..., None]
    px = xT[:, 0] + ego_dx
    py = xT[:, 1] + ego_dy
    dist_t = jnp.sqrt(px * px + py * py)
    ang_t = jnp.arctan2(px, py) % (2.0 * np.pi)
    idx_t = (ang_t / (2.0 * np.pi / PARTITIONS)).astype(jnp.int32)
    valid = (jnp.abs(px + py) > 1e-6) & (dist_t > 0.005)
    t_iota = jax.lax.broadcasted_iota(jnp.int32, (1, OBS, 1), 1)
    code_t = jnp.where(valid, idx_t + PARTITIONS * t_iota, -1)

    code = jnp.transpose(code_t, (0, 2, 1))
    dist = jnp.transpose(dist_t, (0, 2, 1))
    ang = jnp.transpose(ang_t, (0, 2, 1))

    c_iota = jax.lax.broadcasted_iota(jnp.int32, (1, 1, 1, OBS * PARTITIONS), 3)
    M = (code[..., None] == c_iota).astype(f32)
    Mf = M.reshape(bb, NEI * OBS, OBS * PARTITIONS)

    out_re = jax.lax.dot_general(
        Mf, f_re.reshape(bb, NEI * OBS, D // 2),
        (((1,), (1,)), ((0,), (0,))),
        preferred_element_type=f32)

    ones = jnp.ones_like(dist[..., None])
    zeros = jnp.zeros(dist.shape + (5,), f32)
    G = jnp.concatenate([dist[..., None], ang[..., None], ones, zeros],
                        axis=-1)
    out_geo = jax.lax.dot_general(
        Mf, G.reshape(bb, NEI * OBS, 8),
        (((1,), (1,)), ((0,), (0,))),
        preferred_element_type=f32)

    inv_n = 1.0 / (out_geo[..., 2:3] + 0.0001)
    re_part = out_re * inv_n
    d_mean = out_geo[..., 0:1] * inv_n
    a_mean = out_geo[..., 1:2] * inv_n

    wc0 = W_ce_ref[0:1, :].reshape(1, 1, D // 2)
    wc1 = W_ce_ref[1:2, :].reshape(1, 1, D // 2)
    f_pos = jax.nn.relu(d_mean * wc0 + a_mean * wc1
                        + b_ce_ref[...].reshape(1, 1, D // 2))

    re_ref[...] = jnp.concatenate([re_part, f_pos], axis=-1)


@jax.jit
def kernel(x_ego_2d, x_nei_2d, W_te, b_te, W1, b1, W2, b2, W3, b3, W_ce, b_ce):
    grid = (B // BB,)
    x_nei_T = x_nei_2d.transpose(0, 3, 2, 1)
    b_te2 = b_te.reshape(1, D_H)
    b1_2 = b1.reshape(1, D_H)
    b2_2 = b2.reshape(1, D_H)
    b3_2 = b3.reshape(1, D // 2)
    b_ce2 = b_ce.reshape(1, D // 2)

    def rep(shape):
        return pl.BlockSpec(shape, lambda i: (0,) * len(shape))

    f_re, re_flat = pl.pallas_call(
        _fused_kernel,
        grid=grid,
        in_specs=[
            pl.BlockSpec((BB, OBS, 2), lambda i: (i, 0, 0)),
            pl.BlockSpec((BB, NEI, OBS, 2), lambda i: (i, 0, 0, 0)),
            pl.BlockSpec((BB, 2, OBS, NEI), lambda i: (i, 0, 0, 0)),
            rep((2, D_H)), rep((1, D_H)),
            rep((D_H, D_H)), rep((1, D_H)),
            rep((D_H, D_H)), rep((1, D_H)),
            rep((D_H, D // 2)), rep((1, D // 2)),
            rep((2, D // 2)), rep((1, D // 2)),
        ],
        out_specs=[
            pl.BlockSpec((BB, NEI, OBS, D // 2), lambda i: (i, 0, 0, 0)),
            pl.BlockSpec((BB, OBS * PARTITIONS, D), lambda i: (i, 0, 0)),
        ],
        out_shape=[
            jax.ShapeDtypeStruct((B, NEI, OBS, D // 2), jnp.float32),
            jax.ShapeDtypeStruct((B, OBS * PARTITIONS, D), jnp.float32),
        ],
        compiler_params=pltpu.CompilerParams(
            dimension_semantics=("parallel",)),
    )(x_ego_2d, x_nei_2d, x_nei_T, W_te, b_te2, W1, b1_2, W2, b2_2, W3, b3_2,
      W_ce, b_ce2)

    re_matrix = re_flat.reshape(B, OBS, PARTITIONS, D)
    return (re_matrix, f_re)


# speedup vs baseline: 2.0102x; 1.0328x over previous
import jax
import jax.numpy as jnp
import numpy as np
from jax.experimental import pallas as pl
from jax.experimental.pallas import tpu as pltpu

PARTITIONS = 8
D_H = 128
D = 128
OBS = 8
B = 1024
NEI = 64

BB = 16


def _fused_kernel(x_ego_ref, x_nei_ref, x_nei_T_ref, W_te_ref, b_te_ref,
                  W1_ref, b1_ref, W2_ref, b2_ref, W3_ref, b3_ref,
                  W_ce_ref, b_ce_ref, f_re_ref, re_ref):
    bb = x_ego_ref.shape[0]
    f32 = jnp.float32
    x_ego = x_ego_ref[...]
    ego_last = x_ego[:, OBS - 1:OBS, :]

    x_nei = x_nei_ref[...]
    x_nei_pure = x_nei - x_nei[:, :, OBS - 1:OBS, :]
    f_nei = jax.nn.relu(
        jnp.dot(x_nei_pure.reshape(bb * NEI * OBS, 2), W_te_ref[...],
                preferred_element_type=f32) + b_te_ref[...])
    f_nei = f_nei.reshape(bb, NEI, OBS, D_H)

    x_ego_pure = x_ego - ego_last
    f_ego = jax.nn.relu(
        jnp.dot(x_ego_pure.reshape(bb * OBS, 2), W_te_ref[...],
                preferred_element_type=f32) + b_te_ref[...])
    f_ego = f_ego.reshape(bb, 1, OBS, D_H)

    f = (f_ego * f_nei).reshape(bb * NEI * OBS, D_H)

    h = jax.nn.relu(
        jnp.dot(f, W1_ref[...], preferred_element_type=f32) + b1_ref[...])
    h = jax.nn.relu(
        jnp.dot(h, W2_ref[...], preferred_element_type=f32) + b2_ref[...])
    f_re = jax.nn.relu(
        jnp.dot(h, W3_ref[...], preferred_element_type=f32) + b3_ref[...])
    f_re_ref[...] = f_re.reshape(bb, NEI, OBS, D // 2)

    xT = x_nei_T_ref[...]
    egoT = x_ego
    ego_dx = (ego_last[:, :, 0] - egoT[:, :, 0])[..., None]
    ego_dy = (ego_last[:, :, 1] - egoT[:, :, 1])[..., None]
    px = xT[:, 0] + ego_dx
    py = xT[:, 1] + ego_dy
    dist_t = jnp.sqrt(px * px + py * py)
    ang_t = jnp.arctan2(px, py) % (2.0 * np.pi)
    idx_t = (ang_t / (2.0 * np.pi / PARTITIONS)).astype(jnp.int32)
    valid = (jnp.abs(px + py) > 1e-6) & (dist_t > 0.005)
    t_iota = jax.lax.broadcasted_iota(jnp.int32, (1, OBS, 1), 1)
    code_t = jnp.where(valid, idx_t + PARTITIONS * t_iota, -1)

    code = jnp.transpose(code_t, (0, 2, 1))
    dist = jnp.transpose(dist_t, (0, 2, 1))
    ang = jnp.transpose(ang_t, (0, 2, 1))

    c_iota = jax.lax.broadcasted_iota(jnp.int32, (1, 1, 1, OBS * PARTITIONS), 3)
    M = (code[..., None] == c_iota).astype(f32)
    Mf = M.reshape(bb, NEI * OBS, OBS * PARTITIONS)

    out_re = jax.lax.dot_general(
        Mf, f_re.reshape(bb, NEI * OBS, D // 2),
        (((1,), (1,)), ((0,), (0,))),
        preferred_element_type=f32)

    ones = jnp.ones_like(dist[..., None])
    zeros = jnp.zeros(dist.shape + (5,), f32)
    G = jnp.concatenate([dist[..., None], ang[..., None], ones, zeros],
                        axis=-1)
    out_geo = jax.lax.dot_general(
        Mf, G.reshape(bb, NEI * OBS, 8),
        (((1,), (1,)), ((0,), (0,))),
        preferred_element_type=f32)

    inv_n = 1.0 / (out_geo[..., 2:3] + 0.0001)
    re_part = out_re * inv_n
    d_mean = out_geo[..., 0:1] * inv_n
    a_mean = out_geo[..., 1:2] * inv_n

    wc0 = W_ce_ref[0:1, :].reshape(1, 1, D // 2)
    wc1 = W_ce_ref[1:2, :].reshape(1, 1, D // 2)
    f_pos = jax.nn.relu(d_mean * wc0 + a_mean * wc1
                        + b_ce_ref[...].reshape(1, 1, D // 2))

    re_ref[...] = jnp.concatenate([re_part, f_pos], axis=-1)


@jax.jit
def kernel(x_ego_2d, x_nei_2d, W_te, b_te, W1, b1, W2, b2, W3, b3, W_ce, b_ce):
    grid = (B // BB,)
    x_nei_T = x_nei_2d.transpose(0, 3, 2, 1)
    b_te2 = b_te.reshape(1, D_H)
    b1_2 = b1.reshape(1, D_H)
    b2_2 = b2.reshape(1, D_H)
    b3_2 = b3.reshape(1, D // 2)
    b_ce2 = b_ce.reshape(1, D // 2)

    def rep(shape):
        return pl.BlockSpec(shape, lambda i: (0,) * len(shape))

    f_re, re_flat = pl.pallas_call(
        _fused_kernel,
        grid=grid,
        in_specs=[
            pl.BlockSpec((BB, OBS, 2), lambda i: (i, 0, 0)),
            pl.BlockSpec((BB, NEI, OBS, 2), lambda i: (i, 0, 0, 0)),
            pl.BlockSpec((BB, 2, OBS, NEI), lambda i: (i, 0, 0, 0)),
            rep((2, D_H)), rep((1, D_H)),
            rep((D_H, D_H)), rep((1, D_H)),
            rep((D_H, D_H)), rep((1, D_H)),
            rep((D_H, D // 2)), rep((1, D // 2)),
            rep((2, D // 2)), rep((1, D // 2)),
        ],
        out_specs=[
            pl.BlockSpec((BB, NEI, OBS, D // 2), lambda i: (i, 0, 0, 0)),
            pl.BlockSpec((BB, OBS * PARTITIONS, D), lambda i: (i, 0, 0)),
        ],
        out_shape=[
            jax.ShapeDtypeStruct((B, NEI, OBS, D // 2), jnp.float32),
            jax.ShapeDtypeStruct((B, OBS * PARTITIONS, D), jnp.float32),
        ],
        compiler_params=pltpu.CompilerParams(
            dimension_semantics=("parallel",)),
    )(x_ego_2d, x_nei_2d, x_nei_T, W_te, b_te2, W1, b1_2, W2, b2_2, W3, b3_2,
      W_ce, b_ce2)

    re_matrix = re_flat.reshape(B, OBS, PARTITIONS, D)
    return (re_matrix, f_re)
